# baseline (speedup 1.0000x reference)
import functools

import jax
import jax.numpy as jnp
from jax import lax
from jax.experimental import pallas as pl
from jax.experimental.pallas import tpu as pltpu
from jax.experimental.pallas import tpu_sc as plsc

NUM_CLASSES = 1000
BATCH = 1024
SEQ = 26
NUM_CORES = 2
NUM_SUBCORES = 16
NW = NUM_CORES * NUM_SUBCORES
KCH = 40
NKC = NUM_CLASSES // KCH
NU = NKC * SEQ
ITERS = (NU + NW - 1) // NW
GROUPS = BATCH // 16
KDIV_MAGIC = 1639

_mesh = plsc.VectorSubcoreMesh(core_axis_name="c", subcore_axis_name="s")


@functools.partial(
    pl.kernel,
    out_type=jax.ShapeDtypeStruct((SEQ, NUM_CLASSES, BATCH), jnp.float32),
    mesh=_mesh,
    scratch_types=[
        pltpu.VMEM((1, KCH, BATCH), jnp.float32),
        pltpu.VMEM((1, KCH, BATCH), jnp.float32),
        pltpu.VMEM((4, BATCH), jnp.int32),
        pltpu.SemaphoreType.DMA,
        pltpu.SemaphoreType.DMA,
        pltpu.SemaphoreType.DMA,
    ],
    compiler_params=pltpu.CompilerParams(needs_layout_passes=False),
)
def _onehot_sc(xt_hbm, zeros_hbm, out_hbm, buf0, buf1, cols,
               sem0, sem1, csem):
    wid = lax.axis_index("s") * NUM_CORES + lax.axis_index("c")

    def col_s(i):
        u = jnp.minimum(wid + NW * i, NU - 1)
        return u - (u // SEQ) * SEQ

    def col_fetch(i):
        s = col_s(i)
        return pltpu.async_copy(
            xt_hbm.at[pl.ds(s * BATCH, BATCH)], cols.at[i % 4], csem)

    def col_wait(i):
        s = col_s(i)
        pltpu.make_async_copy(
            xt_hbm.at[pl.ds(s * BATCH, BATCH)], cols.at[i % 4], csem).wait()

    col_fetch(0)
    col_fetch(1)
    z0 = pltpu.async_copy(zeros_hbm, buf0, sem0)
    z1 = pltpu.async_copy(zeros_hbm, buf1, sem1)
    z0.wait()
    z1.wait()

    bufs = (buf0, buf1)
    sems = (sem0, sem1)
    lane = lax.iota(jnp.int32, 16)
    ones16 = jnp.full((16,), 1.0, jnp.float32)
    zeros16 = jnp.zeros((16,), jnp.float32)
    zeroidx16 = jnp.zeros((16,), jnp.int32)

    def sweep(buf, set_slot, set_kc, clear_slot, clear_kc):
        def body(g, _):
            blane = g * 16 + lane
            if clear_kc is not None:
                kv = cols[clear_slot, pl.ds(g * 16, 16)]
                kc = (kv * KDIV_MAGIC) >> 16
                rel = kv - kc * KCH
                plsc.store_scatter(buf, [zeroidx16, rel, blane],
                                   zeros16, mask=kc == clear_kc)
            kv = cols[set_slot, pl.ds(g * 16, 16)]
            kc = (kv * KDIV_MAGIC) >> 16
            rel = kv - kc * KCH
            plsc.store_scatter(buf, [zeroidx16, rel, blane],
                               ones16, mask=kc == set_kc)
            return 0

        lax.fori_loop(0, GROUPS, body, 0, unroll=4)

    pending = [None, None]
    pend_info = [None, None]
    for i in range(ITERS):
        b = i % 2
        u = wid + NW * i
        ukc = u // SEQ
        s = u - ukc * SEQ
        last = i == ITERS - 1

        def chunk_body(i=i, b=b, u=u, ukc=ukc, s=s):
            if pending[b] is not None:
                pending[b].wait()
            col_wait(i)
            clear = pend_info[b]
            sweep(bufs[b], i % 4, ukc,
                  None if clear is None else clear[0],
                  None if clear is None else clear[1])
            col_fetch(i + 2)
            for kc in range(NW * i // SEQ, (NW * i + NW - 1) // SEQ + 1):
                if kc >= NKC:
                    continue

                @pl.when(ukc == kc)
                def _(kc=kc, s=s, b=b):
                    dst = out_hbm.at[pl.ds(s, 1), pl.ds(kc * KCH, KCH),
                                     pl.ds(0, BATCH)]
                    pending[b] = pltpu.async_copy(bufs[b], dst, sems[b])

            pend_info[b] = (i % 4, ukc)

        if last and ITERS * NW != NU:
            @pl.when(wid < NU - NW * (ITERS - 1))
            def _():
                chunk_body()
        else:
            chunk_body()

    col_wait(ITERS)
    col_wait(ITERS + 1)
    for b in (0, 1):
        if pending[b] is not None:
            pending[b].wait()


def kernel(x):
    xt = x.astype(jnp.int32).T.reshape(SEQ * BATCH)
    zeros = jnp.zeros((1, KCH, BATCH), jnp.float32)
    y = _onehot_sc(xt, zeros)
    return jnp.transpose(y, (2, 0, 1))


# speedup vs baseline: 1.0867x; 1.0867x over previous
import functools

import jax
import jax.numpy as jnp
from jax import lax
from jax.experimental import pallas as pl
from jax.experimental.pallas import tpu as pltpu
from jax.experimental.pallas import tpu_sc as plsc

NUM_CLASSES = 1000
BATCH = 1024
SEQ = 26
NUM_CORES = 2
NUM_SUBCORES = 16
NW = NUM_CORES * NUM_SUBCORES
KCH = 40
NKC = NUM_CLASSES // KCH
NU = NKC * SEQ
ITERS = (NU + NW - 1) // NW
GROUPS = BATCH // 16
KDIV_MAGIC = 1639

_mesh = plsc.VectorSubcoreMesh(core_axis_name="c", subcore_axis_name="s")


@functools.partial(
    pl.kernel,
    out_type=jax.ShapeDtypeStruct((SEQ, NUM_CLASSES, BATCH), jnp.float32),
    mesh=_mesh,
    scratch_types=[
        pltpu.VMEM((1, KCH, BATCH), jnp.float32),
        pltpu.VMEM((1, KCH, BATCH), jnp.float32),
        pltpu.VMEM((SEQ * BATCH,), jnp.int32),
        pltpu.SemaphoreType.DMA,
        pltpu.SemaphoreType.DMA,
    ],
    compiler_params=pltpu.CompilerParams(needs_layout_passes=False),
)
def _onehot_sc(xt_hbm, zeros_hbm, out_hbm, buf0, buf1, xt_v, sem0, sem1):
    wid = lax.axis_index("s") * NUM_CORES + lax.axis_index("c")

    z0 = pltpu.async_copy(zeros_hbm, buf0, sem0)
    z1 = pltpu.async_copy(zeros_hbm, buf1, sem1)
    pltpu.sync_copy(xt_hbm, xt_v)
    z0.wait()
    z1.wait()

    bufs = (buf0, buf1)
    sems = (sem0, sem1)
    lane = lax.iota(jnp.int32, 16)
    ones16 = jnp.full((16,), 1.0, jnp.float32)
    zeros16 = jnp.zeros((16,), jnp.float32)
    zeroidx16 = jnp.zeros((16,), jnp.int32)

    def sweep(buf, set_s, set_kc, clear_info):
        set_base = set_s * BATCH

        def scatter_one(base, g, want_kc, value16):
            kv = xt_v[pl.ds(base + g * 16, 16)]
            kc = (kv * KDIV_MAGIC) >> 16
            rel = kv - kc * KCH
            plsc.store_scatter(buf, [zeroidx16, rel, g * 16 + lane],
                               value16, mask=kc == want_kc)

        def body(g, _):
            if clear_info is not None:
                scatter_one(clear_info[0] * BATCH, g, clear_info[1],
                            zeros16)
            scatter_one(set_base, g, set_kc, ones16)
            return 0

        lax.fori_loop(0, GROUPS, body, 0, unroll=4)

    pending = [None, None]
    pend_info = [None, None]
    for i in range(ITERS):
        b = i % 2
        u = wid + NW * i
        ukc = u // SEQ
        s = u - ukc * SEQ

        def chunk_body(i=i, b=b, u=u, ukc=ukc, s=s):
            if pending[b] is not None:
                pending[b].wait()
            sweep(bufs[b], s, ukc, pend_info[b])
            for kc in range(NW * i // SEQ, (NW * i + NW - 1) // SEQ + 1):
                if kc >= NKC:
                    continue

                @pl.when(ukc == kc)
                def _(kc=kc, s=s, b=b):
                    dst = out_hbm.at[pl.ds(s, 1), pl.ds(kc * KCH, KCH),
                                     pl.ds(0, BATCH)]
                    pending[b] = pltpu.async_copy(bufs[b], dst, sems[b])

            pend_info[b] = (s, ukc)

        if i == ITERS - 1 and ITERS * NW != NU:
            @pl.when(wid < NU - NW * (ITERS - 1))
            def _():
                chunk_body()
        else:
            chunk_body()

    for b in (0, 1):
        if pending[b] is not None:
            pending[b].wait()


def kernel(x):
    xt = x.astype(jnp.int32).T.reshape(SEQ * BATCH)
    zeros = jnp.zeros((1, KCH, BATCH), jnp.float32)
    y = _onehot_sc(xt, zeros)
    return jnp.transpose(y, (2, 0, 1))


# speedup vs baseline: 1.2373x; 1.1386x over previous
import functools

import jax
import jax.numpy as jnp
from jax import lax
from jax.experimental import pallas as pl
from jax.experimental.pallas import tpu as pltpu
from jax.experimental.pallas import tpu_sc as plsc

NUM_CLASSES = 1000
BATCH = 1024
SEQ = 26
NUM_CORES = 2
NUM_SUBCORES = 16
KCH = 40
NKC = NUM_CLASSES // KCH
GROUPS = BATCH // 16
BUFGROUPS = KCH * BATCH // 16
KDIV_MAGIC = 1639

_mesh = plsc.VectorSubcoreMesh(core_axis_name="c", subcore_axis_name="s")


@functools.partial(
    pl.kernel,
    out_type=jax.ShapeDtypeStruct((SEQ, NUM_CLASSES, BATCH), jnp.float32),
    mesh=_mesh,
    scratch_types=[
        pltpu.VMEM((1, KCH, BATCH), jnp.float32),
        pltpu.VMEM((1, KCH, BATCH), jnp.float32),
        pltpu.VMEM((BATCH,), jnp.int32),
        pltpu.VMEM((BATCH,), jnp.int32),
        pltpu.VMEM((BATCH,), jnp.int32),
        pltpu.SemaphoreType.DMA,
        pltpu.SemaphoreType.DMA,
    ],
    compiler_params=pltpu.CompilerParams(
        needs_layout_passes=False,
        skip_device_barrier=True,
        disable_bounds_checks=True,
        disable_semaphore_checks=True,
    ),
)
def _onehot_sc(xt_hbm, zeros_hbm, out_hbm,
               buf0, buf1, col, kcv, relv, sem0, sem1):
    wid = lax.axis_index("s") * NUM_CORES + lax.axis_index("c")

    @pl.when(wid < SEQ)
    def _():
        s = wid
        z0 = pltpu.async_copy(zeros_hbm, buf0, sem0)
        pltpu.sync_copy(xt_hbm.at[pl.ds(s * BATCH, BATCH)], col)

        lane = lax.iota(jnp.int32, 16)
        ones16 = jnp.full((16,), 1.0, jnp.float32)
        zeros16 = jnp.zeros((16,), jnp.float32)
        zeroidx16 = jnp.zeros((16,), jnp.int32)

        def zero_fill(g, _):
            buf1[0, g >> 6, pl.ds((g & 63) * 16, 16)] = zeros16
            return 0

        lax.fori_loop(0, BUFGROUPS, zero_fill, 0, unroll=8)

        def bucket(g, _):
            kv = col[pl.ds(g * 16, 16)]
            kc = (kv * KDIV_MAGIC) >> 16
            kcv[pl.ds(g * 16, 16)] = kc
            relv[pl.ds(g * 16, 16)] = kv - kc * KCH
            return 0

        lax.fori_loop(0, GROUPS, bucket, 0, unroll=4)
        z0.wait()

        bufs = (buf0, buf1)
        sems = (sem0, sem1)

        def sweep(buf, set_kc, clear_kc):
            def body(g, _):
                kc = kcv[pl.ds(g * 16, 16)]
                rel = relv[pl.ds(g * 16, 16)]
                blane = g * 16 + lane
                if clear_kc is not None:
                    plsc.store_scatter(buf, [zeroidx16, rel, blane],
                                       zeros16, mask=kc == clear_kc)
                plsc.store_scatter(buf, [zeroidx16, rel, blane],
                                   ones16, mask=kc == set_kc)
                return 0

            lax.fori_loop(0, GROUPS, body, 0, unroll=4)

        pending = [None, None]
        for kc in range(NKC):
            b = kc % 2
            if pending[b] is not None:
                pending[b].wait()
            sweep(bufs[b], kc, kc - 2 if kc >= 2 else None)
            dst = out_hbm.at[pl.ds(s, 1), pl.ds(kc * KCH, KCH),
                             pl.ds(0, BATCH)]
            pending[b] = pltpu.async_copy(bufs[b], dst, sems[b])
        for b in (0, 1):
            pending[b].wait()


def kernel(x):
    xt = x.astype(jnp.int32).T.reshape(SEQ * BATCH)
    zeros = jnp.zeros((1, KCH, BATCH), jnp.float32)
    y = _onehot_sc(xt, zeros)
    return jnp.transpose(y, (2, 0, 1))
